# Initial kernel scaffold; baseline (speedup 1.0000x reference)
import jax
import jax.numpy as jnp
from jax.experimental import pallas as pl


def kernel(x, mstream, varstream):
    raise NotImplementedError("write your pallas kernel here")


# speedup vs baseline: 78.8011x; 78.8011x over previous
import functools

import jax
import jax.numpy as jnp
import numpy as np
from jax.experimental import pallas as pl
from jax.experimental.pallas import tpu as pltpu

_A = 0.999
_OMA = 1.0 - _A
_EPS = 1e-05
_C = 256
_PAD = 8


@functools.lru_cache(maxsize=None)
def _chunk_mats(C: int):
    j = np.arange(C, dtype=np.float64)[:, None]
    k = np.arange(C, dtype=np.float64)[None, :]
    L = np.where(k < j, _A ** np.maximum(j - 1 - k, 0.0), 0.0)
    Tm = np.zeros((C, C + _PAD), np.float32)
    Tv = np.zeros((C, C + _PAD), np.float32)
    Tm[:, 0] = _A ** np.arange(C)
    Tv[:, 0] = _A ** np.arange(C)
    Tm[:, _PAD:] = _OMA * L
    Tv[:, _PAD:] = L
    return jnp.asarray(Tm), jnp.asarray(Tv)


def _body(x_ref, m0_ref, v0_ref, tm_ref, tv_ref,
          out_ref, mout_ref, vout_ref, rm_ref, rv_ref):
    b = pl.program_id(1)

    @pl.when(b == 0)
    def _init():
        rm_ref[0:_PAD, :] = jnp.zeros_like(rm_ref[0:_PAD, :])
        rv_ref[0:_PAD, :] = jnp.zeros_like(rv_ref[0:_PAD, :])
        rm_ref[0:1, :] = m0_ref[...]
        rv_ref[0:1, :] = v0_ref[...]

    xb = x_ref[...]
    rm_ref[_PAD:, :] = xb
    m = jax.lax.dot_general(
        tm_ref[...], rm_ref[...], (((1,), (0,)), ((), ())),
        precision=jax.lax.Precision.HIGHEST,
        preferred_element_type=jnp.float32)
    d = xb - m
    bb = (_A * _OMA) * (d * d)
    rv_ref[_PAD:, :] = bb
    v = jax.lax.dot_general(
        tv_ref[...], rv_ref[...], (((1,), (0,)), ((), ())),
        precision=jax.lax.Precision.HIGHEST,
        preferred_element_type=jnp.float32)
    out_ref[...] = d * jax.lax.rsqrt(v + _EPS)

    d_last = d[_C - 1:_C, :]
    rm_ref[0:1, :] = m[_C - 1:_C, :] + _OMA * d_last
    rv_ref[0:1, :] = _A * v[_C - 1:_C, :] + (_A * _OMA) * (d_last * d_last)
    mout_ref[...] = rm_ref[0:1, :]
    vout_ref[...] = rv_ref[0:1, :]


def kernel(x, mstream, varstream):
    B, F = x.shape
    C = _C
    Fb = F // 2 if F % 256 == 0 and F >= 512 else F
    nb = B // C
    nf = F // Fb
    Tm, Tv = _chunk_mats(C)
    m2 = mstream.reshape(1, F)
    v2 = varstream.reshape(1, F)

    out, mfin, vfin = pl.pallas_call(
        _body,
        grid=(nf, nb),
        in_specs=[
            pl.BlockSpec((C, Fb), lambda f, b: (b, f)),
            pl.BlockSpec((1, Fb), lambda f, b: (0, f)),
            pl.BlockSpec((1, Fb), lambda f, b: (0, f)),
            pl.BlockSpec((C, C + _PAD), lambda f, b: (0, 0)),
            pl.BlockSpec((C, C + _PAD), lambda f, b: (0, 0)),
        ],
        out_specs=[
            pl.BlockSpec((C, Fb), lambda f, b: (b, f)),
            pl.BlockSpec((1, Fb), lambda f, b: (0, f)),
            pl.BlockSpec((1, Fb), lambda f, b: (0, f)),
        ],
        out_shape=[
            jax.ShapeDtypeStruct((B, F), jnp.float32),
            jax.ShapeDtypeStruct((1, F), jnp.float32),
            jax.ShapeDtypeStruct((1, F), jnp.float32),
        ],
        scratch_shapes=[
            pltpu.VMEM((C + _PAD, Fb), jnp.float32),
            pltpu.VMEM((C + _PAD, Fb), jnp.float32),
        ],
        compiler_params=pltpu.CompilerParams(
            dimension_semantics=("parallel", "arbitrary")),
    )(x, m2, v2, Tm, Tv)
    return out, mfin.reshape(F), vfin.reshape(F)


# speedup vs baseline: 235.7285x; 2.9914x over previous
import functools

import jax
import jax.numpy as jnp
import numpy as np
from jax.experimental import pallas as pl
from jax.experimental.pallas import tpu as pltpu

_A = 0.999
_OMA = 1.0 - _A
_EPS = 1e-05
_C = 256
_PAD = 8


@functools.lru_cache(maxsize=None)
def _chunk_mats(C: int):
    j = np.arange(C, dtype=np.float64)[:, None]
    k = np.arange(C, dtype=np.float64)[None, :]
    L = np.where(k < j, _A ** np.maximum(j - 1 - k, 0.0), 0.0)
    Tm = np.zeros((C, C + _PAD), np.float32)
    Tv = np.zeros((C, C + _PAD), np.float32)
    Tm[:, 0] = _A ** np.arange(C)
    Tv[:, 0] = _A ** np.arange(C)
    Tm[:, 1] = Tm[:, 0]
    Tv[:, 1] = Tv[:, 0]
    Tm[:, _PAD:] = _OMA * L
    Tv[:, _PAD:] = L
    return jnp.asarray(Tm), jnp.asarray(Tv)


def _store_carry(ref, val):
    hi = val.astype(jnp.bfloat16).astype(jnp.float32)
    ref[0:1, :] = hi
    ref[1:2, :] = val - hi


def _body(x_ref, m0_ref, v0_ref, tm_ref, tv_ref,
          out_ref, mout_ref, vout_ref, rm_ref, rv_ref):
    b = pl.program_id(1)

    @pl.when(b == 0)
    def _init():
        rm_ref[0:_PAD, :] = jnp.zeros_like(rm_ref[0:_PAD, :])
        rv_ref[0:_PAD, :] = jnp.zeros_like(rv_ref[0:_PAD, :])
        _store_carry(rm_ref, m0_ref[...])
        _store_carry(rv_ref, v0_ref[...])

    xb = x_ref[...]
    rm_ref[_PAD:, :] = xb
    m = jax.lax.dot_general(
        tm_ref[...], rm_ref[...], (((1,), (0,)), ((), ())),
        precision=jax.lax.Precision.DEFAULT,
        preferred_element_type=jnp.float32)
    d = xb - m
    bb = (_A * _OMA) * (d * d)
    rv_ref[_PAD:, :] = bb
    v = jax.lax.dot_general(
        tv_ref[...], rv_ref[...], (((1,), (0,)), ((), ())),
        precision=jax.lax.Precision.DEFAULT,
        preferred_element_type=jnp.float32)
    out_ref[...] = d * jax.lax.rsqrt(v + _EPS)

    d_last = d[_C - 1:_C, :]
    m_carry = m[_C - 1:_C, :] + _OMA * d_last
    v_carry = _A * v[_C - 1:_C, :] + (_A * _OMA) * (d_last * d_last)
    _store_carry(rm_ref, m_carry)
    _store_carry(rv_ref, v_carry)
    mout_ref[...] = m_carry
    vout_ref[...] = v_carry


def kernel(x, mstream, varstream):
    B, F = x.shape
    C = _C
    Fb = F // 2 if F % 256 == 0 and F >= 512 else F
    nb = B // C
    nf = F // Fb
    Tm, Tv = _chunk_mats(C)
    m2 = mstream.reshape(1, F)
    v2 = varstream.reshape(1, F)

    out, mfin, vfin = pl.pallas_call(
        _body,
        grid=(nf, nb),
        in_specs=[
            pl.BlockSpec((C, Fb), lambda f, b: (b, f)),
            pl.BlockSpec((1, Fb), lambda f, b: (0, f)),
            pl.BlockSpec((1, Fb), lambda f, b: (0, f)),
            pl.BlockSpec((C, C + _PAD), lambda f, b: (0, 0)),
            pl.BlockSpec((C, C + _PAD), lambda f, b: (0, 0)),
        ],
        out_specs=[
            pl.BlockSpec((C, Fb), lambda f, b: (b, f)),
            pl.BlockSpec((1, Fb), lambda f, b: (0, f)),
            pl.BlockSpec((1, Fb), lambda f, b: (0, f)),
        ],
        out_shape=[
            jax.ShapeDtypeStruct((B, F), jnp.float32),
            jax.ShapeDtypeStruct((1, F), jnp.float32),
            jax.ShapeDtypeStruct((1, F), jnp.float32),
        ],
        scratch_shapes=[
            pltpu.VMEM((C + _PAD, Fb), jnp.float32),
            pltpu.VMEM((C + _PAD, Fb), jnp.float32),
        ],
        compiler_params=pltpu.CompilerParams(
            dimension_semantics=("parallel", "arbitrary")),
    )(x, m2, v2, Tm, Tv)
    return out, mfin.reshape(F), vfin.reshape(F)
